# baseline (speedup 1.0000x reference)
import jax
import jax.numpy as jnp
from jax import lax
from jax.experimental import pallas as pl
from jax.experimental.pallas import tpu as pltpu

BATCH_HEADS = 32
SEQ = 4096
D_K = 64
ROUNDS = 4
NB2 = 32
CAT = 2 * NB2

S_BLK = 4096
B_BLK = 2


def _lsh_kernel(x_ref, w_ref, o_ref):
    for i in range(B_BLK):
        x = x_ref[i]
        w = w_ref[i]
        ht = lax.dot_general(w, x, (((0,), (1,)), ((), ())),
                             preferred_element_type=jnp.float32)
        packed = jnp.zeros((S_BLK,), jnp.int32)
        for r in range(ROUNDS):
            g = ht[r * NB2:(r + 1) * NB2, :]
            cat = jnp.concatenate([g, -g], axis=0)
            idx = jnp.argmax(cat, axis=0).astype(jnp.int32)
            packed = packed | (idx << (8 * r))
        o_ref[i, 0] = packed


@jax.jit
def kernel(inp, rand_matrix):
    w2 = rand_matrix.reshape(BATCH_HEADS, D_K, ROUNDS * NB2)
    packed = pl.pallas_call(
        _lsh_kernel,
        grid=(BATCH_HEADS // B_BLK,),
        in_specs=[
            pl.BlockSpec((B_BLK, S_BLK, D_K), lambda b: (b, 0, 0)),
            pl.BlockSpec((B_BLK, D_K, ROUNDS * NB2), lambda b: (b, 0, 0)),
        ],
        out_specs=pl.BlockSpec((B_BLK, 1, S_BLK), lambda b: (b, 0, 0)),
        out_shape=jax.ShapeDtypeStruct((BATCH_HEADS, 1, SEQ), jnp.int32),
        compiler_params=pltpu.CompilerParams(
            dimension_semantics=("parallel",),
        ),
    )(inp, w2)
    out8 = lax.bitcast_convert_type(packed.reshape(BATCH_HEADS, SEQ), jnp.uint8)
    return out8.astype(jnp.int32)


# speedup vs baseline: 1.0629x; 1.0629x over previous
import jax
import jax.numpy as jnp
from jax import lax
from jax.experimental import pallas as pl
from jax.experimental.pallas import tpu as pltpu

BATCH_HEADS = 32
SEQ = 4096
D_K = 64
ROUNDS = 4
NB2 = 32
CAT = 2 * NB2

S_BLK = 4096
B_BLK = 4


def _lsh_kernel(x_ref, w_ref, o_ref):
    for i in range(B_BLK):
        x = x_ref[i]
        w = w_ref[i]
        ht = lax.dot_general(w, x, (((0,), (1,)), ((), ())),
                             preferred_element_type=jnp.float32)
        packed = jnp.zeros((S_BLK,), jnp.int32)
        for r in range(ROUNDS):
            g = ht[r * CAT:(r + 1) * CAT, :]
            idx = jnp.argmax(g, axis=0).astype(jnp.int32)
            packed = packed | (idx << (8 * r))
        o_ref[i, 0] = packed


@jax.jit
def kernel(inp, rand_matrix):
    w = rand_matrix
    wcat = jnp.concatenate([w, -w], axis=-1)
    wcat = wcat.reshape(BATCH_HEADS, D_K, ROUNDS * CAT)
    packed = pl.pallas_call(
        _lsh_kernel,
        grid=(BATCH_HEADS // B_BLK,),
        in_specs=[
            pl.BlockSpec((B_BLK, S_BLK, D_K), lambda b: (b, 0, 0)),
            pl.BlockSpec((B_BLK, D_K, ROUNDS * CAT), lambda b: (b, 0, 0)),
        ],
        out_specs=pl.BlockSpec((B_BLK, 1, S_BLK), lambda b: (b, 0, 0)),
        out_shape=jax.ShapeDtypeStruct((BATCH_HEADS, 1, SEQ), jnp.int32),
        compiler_params=pltpu.CompilerParams(
            dimension_semantics=("parallel",),
        ),
    )(inp, wcat)
    out8 = lax.bitcast_convert_type(packed.reshape(BATCH_HEADS, SEQ), jnp.uint8)
    return out8.astype(jnp.int32)
